# Initial kernel scaffold; baseline (speedup 1.0000x reference)
import jax
import jax.numpy as jnp
from jax.experimental import pallas as pl


def kernel(x, edge_index, edge_vals, W1, b1, W2, b2, W3, b3):
    raise NotImplementedError("write your pallas kernel here")


# speedup vs baseline: 3.6949x; 3.6949x over previous
import functools

import jax
import jax.numpy as jnp
from jax import lax
from jax.experimental import pallas as pl
from jax.experimental.pallas import tpu as pltpu
from jax.experimental.pallas import tpu_sc as plsc

_NC = 2
_NS = 16
_L = 16


def _mm_body(x_ref, w_ref, b_ref, o_ref):
    o_ref[...] = (
        jnp.dot(x_ref[...], w_ref[...], preferred_element_type=jnp.float32)
        + b_ref[...]
    )


def _mm(x, W, b, bm=2000):
    n, d = x.shape
    f = W.shape[1]
    return pl.pallas_call(
        _mm_body,
        grid=(n // bm,),
        in_specs=[
            pl.BlockSpec((bm, d), lambda i: (i, 0)),
            pl.BlockSpec((d, f), lambda i: (0, 0)),
            pl.BlockSpec((1, f), lambda i: (0, 0)),
        ],
        out_specs=pl.BlockSpec((bm, f), lambda i: (i, 0)),
        out_shape=jax.ShapeDtypeStruct((n, f), jnp.float32),
    )(x, W, b.reshape(1, f))


def _mm_fused_body(p_ref, w_ref, b_ref, o_ref):
    h = jax.nn.relu(p_ref[0] + p_ref[1])
    o_ref[...] = (
        jnp.dot(h, w_ref[...], preferred_element_type=jnp.float32) + b_ref[...]
    )


def _mm_fused(parts, W, b, bm=2000):
    _, n, d = parts.shape
    f = W.shape[1]
    return pl.pallas_call(
        _mm_fused_body,
        grid=(n // bm,),
        in_specs=[
            pl.BlockSpec((2, bm, d), lambda i: (0, i, 0)),
            pl.BlockSpec((d, f), lambda i: (0, 0)),
            pl.BlockSpec((1, f), lambda i: (0, 0)),
        ],
        out_specs=pl.BlockSpec((bm, f), lambda i: (i, 0)),
        out_shape=jax.ShapeDtypeStruct((n, f), jnp.float32),
    )(parts, W, b.reshape(1, f))


def _make_add_body(f_out):
    def _add_body(p_ref, o_ref):
        o_ref[...] = p_ref[0, :, :f_out] + p_ref[1, :, :f_out]
    return _add_body


def _add_parts(parts, f_out, bm=2000):
    _, n, f = parts.shape
    return pl.pallas_call(
        _make_add_body(f_out),
        grid=(n // bm,),
        in_specs=[pl.BlockSpec((2, bm, f), lambda i: (0, i, 0))],
        out_specs=pl.BlockSpec((bm, f_out), lambda i: (i, 0)),
        out_shape=jax.ShapeDtypeStruct((n, f_out), jnp.float32),
    )(parts)


@functools.lru_cache(maxsize=None)
def _make_scatter(n, e, f, b=80):
    nw = _NC * _NS
    epw = e // nw
    nz = n // b
    assert epw % b == 0 and b % _L == 0
    assert n % b == 0 and b % 8 == 0 and f % _L == 0

    mesh = plsc.VectorSubcoreMesh(core_axis_name="c", subcore_axis_name="s")

    @functools.partial(
        pl.kernel,
        out_type=jax.ShapeDtypeStruct((_NC, n, f), jnp.float32),
        mesh=mesh,
        scratch_types=[
            pltpu.VMEM_SHARED((n, f), jnp.float32),
            pltpu.VMEM((b,), jnp.int32),
            pltpu.VMEM((b,), jnp.int32),
            pltpu.VMEM((b,), jnp.float32),
            pltpu.VMEM((b, f), jnp.float32),
            pltpu.SemaphoreType.DMA,
        ],
    )
    def scatter_kernel(h_hbm, src_hbm, dst_hbm, vals_hbm, out_hbm,
                       acc, src_v, dst_v, vals_v, rows_v, sem):
        c = lax.axis_index("c")
        s = lax.axis_index("s")
        wid = c * _NS + s

        zeros = jnp.zeros((_L,), jnp.float32)

        def zrow(i, carry):
            for j in range(f // _L):
                rows_v[i, pl.ds(j * _L, _L)] = zeros
            return carry

        lax.fori_loop(0, b, zrow, 0)
        for r in range((nz + _NS - 1) // _NS):
            kc = r * _NS + s

            @pl.when(kc < nz)
            def _():
                pltpu.sync_copy(rows_v, acc.at[pl.ds(kc * b, b)])

        plsc.subcore_barrier()

        ebase = wid * epw

        def chunk(k, carry):
            eo = ebase + k * b
            pltpu.sync_copy(src_hbm.at[pl.ds(eo, b)], src_v)
            pltpu.sync_copy(dst_hbm.at[pl.ds(eo, b)], dst_v)
            pltpu.sync_copy(vals_hbm.at[pl.ds(eo, b)], vals_v)
            pltpu.async_copy(h_hbm.at[src_v], rows_v, sem).wait()

            def scale(i, carry2):
                vvec = vals_v[pl.ds(i * _L, _L)]
                for j in range(_L):
                    row = i * _L + j
                    v = vvec[j]
                    for jf in range(f // _L):
                        sl = pl.ds(jf * _L, _L)
                        rows_v[row, sl] = rows_v[row, sl] * v
                return carry2

            lax.fori_loop(0, b // _L, scale, 0)
            pltpu.sync_copy(rows_v, acc.at[dst_v], add=True)
            return carry

        lax.fori_loop(0, epw // b, chunk, 0)
        plsc.subcore_barrier()

        for r in range((nz + _NS - 1) // _NS):
            kc = r * _NS + s

            @pl.when(kc < nz)
            def _():
                pltpu.sync_copy(acc.at[pl.ds(kc * b, b)],
                                out_hbm.at[c, pl.ds(kc * b, b)])

    return scatter_kernel


def kernel(x, edge_index, edge_vals, W1, b1, W2, b2, W3, b3):
    n = x.shape[0]
    e = edge_vals.shape[0]
    h = W2.shape[0]
    c_out = W3.shape[1]
    src = edge_index[1]
    dst = edge_index[0]

    scatter_h = _make_scatter(n, e, h)

    W3p = jnp.pad(W3, ((0, 0), (0, h - c_out)))
    b3p = jnp.pad(b3, (0, h - c_out))

    a1 = _mm(x, W1, b1)
    p1 = scatter_h(a1, src, dst, edge_vals)
    a2 = _mm_fused(p1, W2, b2)
    p2 = scatter_h(a2, src, dst, edge_vals)
    a3 = _mm_fused(p2, W3p, b3p)
    p3 = scatter_h(a3, src, dst, edge_vals)
    return _add_parts(p3, c_out)


# speedup vs baseline: 9.8421x; 2.6637x over previous
import functools

import jax
import jax.numpy as jnp
from jax import lax
from jax.experimental import pallas as pl
from jax.experimental.pallas import tpu as pltpu
from jax.experimental.pallas import tpu_sc as plsc

_NC = 2
_NS = 16
_L = 16


def _mm_body(x_ref, w_ref, b_ref, o_ref):
    o_ref[...] = (
        jnp.dot(x_ref[...], w_ref[...], preferred_element_type=jnp.float32)
        + b_ref[...]
    )


def _mm(x, W, b, bm=2000):
    n, d = x.shape
    f = W.shape[1]
    return pl.pallas_call(
        _mm_body,
        grid=(n // bm,),
        in_specs=[
            pl.BlockSpec((bm, d), lambda i: (i, 0)),
            pl.BlockSpec((d, f), lambda i: (0, 0)),
            pl.BlockSpec((1, f), lambda i: (0, 0)),
        ],
        out_specs=pl.BlockSpec((bm, f), lambda i: (i, 0)),
        out_shape=jax.ShapeDtypeStruct((n, f), jnp.float32),
    )(x, W, b.reshape(1, f))


def _mm_fused_body(p_ref, w_ref, b_ref, o_ref):
    h = jax.nn.relu(p_ref[0] + p_ref[1])
    o_ref[...] = (
        jnp.dot(h, w_ref[...], preferred_element_type=jnp.float32) + b_ref[...]
    )


def _mm_fused(parts, W, b, bm=2000):
    _, n, d = parts.shape
    f = W.shape[1]
    return pl.pallas_call(
        _mm_fused_body,
        grid=(n // bm,),
        in_specs=[
            pl.BlockSpec((2, bm, d), lambda i: (0, i, 0)),
            pl.BlockSpec((d, f), lambda i: (0, 0)),
            pl.BlockSpec((1, f), lambda i: (0, 0)),
        ],
        out_specs=pl.BlockSpec((bm, f), lambda i: (i, 0)),
        out_shape=jax.ShapeDtypeStruct((n, f), jnp.float32),
    )(parts, W, b.reshape(1, f))


def _make_add_body(f_out):
    def _add_body(p_ref, o_ref):
        o_ref[...] = p_ref[0, :, :f_out] + p_ref[1, :, :f_out]
    return _add_body


def _add_parts(parts, f_out, bm=2000):
    _, n, f = parts.shape
    return pl.pallas_call(
        _make_add_body(f_out),
        grid=(n // bm,),
        in_specs=[pl.BlockSpec((2, bm, f), lambda i: (0, i, 0))],
        out_specs=pl.BlockSpec((bm, f_out), lambda i: (i, 0)),
        out_shape=jax.ShapeDtypeStruct((n, f_out), jnp.float32),
    )(parts)


@functools.lru_cache(maxsize=None)
def _make_scatter(n, e, f, b=80, nb=3):
    nw = _NC * _NS
    epw = e // nw
    nc_ = epw // b
    nz = n // b
    assert epw % b == 0 and b % _L == 0 and nc_ >= nb + 1
    assert n % b == 0 and b % 8 == 0 and f % _L == 0

    mesh = plsc.VectorSubcoreMesh(core_axis_name="c", subcore_axis_name="s")

    @functools.partial(
        pl.kernel,
        out_type=jax.ShapeDtypeStruct((_NC, n, f), jnp.float32),
        mesh=mesh,
        scratch_types=(
            [pltpu.VMEM_SHARED((n, f), jnp.float32)]
            + [pltpu.VMEM((2, b), jnp.int32) for _ in range(nb)]
            + [pltpu.VMEM((b,), jnp.float32) for _ in range(nb)]
            + [pltpu.VMEM((b, f), jnp.float32) for _ in range(nb)]
            + [pltpu.SemaphoreType.DMA for _ in range(4 * nb)]
        ),
    )
    def scatter_kernel(h_hbm, pk_hbm, vals_hbm, out_hbm, acc, *bufs):
        pks = bufs[:nb]
        vals = bufs[nb:2 * nb]
        rows = bufs[2 * nb:3 * nb]
        psem = bufs[3 * nb:4 * nb]
        vsem = bufs[4 * nb:5 * nb]
        gsem = bufs[5 * nb:6 * nb]
        ssem = bufs[6 * nb:7 * nb]
        c = lax.axis_index("c")
        s = lax.axis_index("s")
        wid = c * _NS + s
        cbase = wid * nc_

        def start_pk(ci, s_):
            pltpu.async_copy(pk_hbm.at[cbase + ci], pks[s_], psem[s_])
            pltpu.async_copy(vals_hbm.at[pl.ds((cbase + ci) * b, b)],
                             vals[s_], vsem[s_])

        def wait_pk(s_):
            pltpu.make_async_copy(pk_hbm.at[0], pks[s_], psem[s_]).wait()
            pltpu.make_async_copy(vals_hbm.at[pl.ds(0, b)], vals[s_],
                                  vsem[s_]).wait()

        def start_gather(s_):
            pltpu.async_copy(h_hbm.at[pks[s_].at[0]], rows[s_], gsem[s_])

        def wait_gather(s_):
            pltpu.make_async_copy(h_hbm.at[pks[s_].at[0]], rows[s_],
                                  gsem[s_]).wait()

        def start_scatter(s_):
            pltpu.async_copy(rows[s_], acc.at[pks[s_].at[1]], ssem[s_],
                             add=True)

        def wait_scatter(s_):
            pltpu.make_async_copy(rows[s_], acc.at[pks[s_].at[1]],
                                  ssem[s_]).wait()

        def scale(s_):
            vals_ref = vals[s_]
            rows_ref = rows[s_]

            def body(g, carry):
                vvec = vals_ref[pl.ds(g * _L, _L)]
                for t in range(_L):
                    row = g * _L + t
                    v = vvec[t]
                    for jf in range(f // _L):
                        sl = pl.ds(jf * _L, _L)
                        rows_ref[row, sl] = rows_ref[row, sl] * v
                return carry

            lax.fori_loop(0, b // _L, body, 0)

        zeros = jnp.zeros((_L,), jnp.float32)

        def zrow(i, carry):
            for j in range(f // _L):
                rows[0][i, pl.ds(j * _L, _L)] = zeros
            return carry

        lax.fori_loop(0, b, zrow, 0)
        for r in range((nz + _NS - 1) // _NS):
            kc = r * _NS + s

            @pl.when(kc < nz)
            def _():
                pltpu.sync_copy(rows[0], acc.at[pl.ds(kc * b, b)])

        plsc.subcore_barrier()

        start_pk(0, 0)
        start_pk(1, 1)
        wait_pk(0)
        start_gather(0)

        def triple(i, carry):
            for j in range(nb):
                ci = i * nb + j

                @pl.when((ci + 2 < nc_) & (ci >= 1))
                def _():
                    wait_scatter((j + 2) % nb)

                @pl.when(ci + 2 < nc_)
                def _():
                    start_pk(ci + 2, (j + 2) % nb)

                @pl.when(ci + 1 < nc_)
                def _():
                    wait_pk((j + 1) % nb)
                    start_gather((j + 1) % nb)

                @pl.when(ci < nc_)
                def _():
                    wait_gather(j)
                    scale(j)
                    start_scatter(j)
            return carry

        lax.fori_loop(0, (nc_ + nb - 1) // nb, triple, 0)
        for j in range(nb):
            wait_scatter(j)
        plsc.subcore_barrier()

        for r in range((nz + _NS - 1) // _NS):
            kc = r * _NS + s

            @pl.when(kc < nz)
            def _():
                pltpu.sync_copy(acc.at[pl.ds(kc * b, b)],
                                out_hbm.at[c, pl.ds(kc * b, b)])

    return scatter_kernel


def kernel(x, edge_index, edge_vals, W1, b1, W2, b2, W3, b3):
    n = x.shape[0]
    e = edge_vals.shape[0]
    h = W2.shape[0]
    c_out = W3.shape[1]
    src = edge_index[1]
    dst = edge_index[0]

    b = 80
    g = e // b
    packed = jnp.stack([src.reshape(g, b), dst.reshape(g, b)], axis=1)

    scatter_h = _make_scatter(n, e, h, b=b)

    W3p = jnp.pad(W3, ((0, 0), (0, h - c_out)))
    b3p = jnp.pad(b3, (0, h - c_out))

    a1 = _mm(x, W1, b1)
    p1 = scatter_h(a1, packed, edge_vals)
    a2 = _mm_fused(p1, W2, b2)
    p2 = scatter_h(a2, packed, edge_vals)
    a3 = _mm_fused(p2, W3p, b3p)
    p3 = scatter_h(a3, packed, edge_vals)
    return _add_parts(p3, c_out)


# speedup vs baseline: 11.3741x; 1.1557x over previous
import functools

import jax
import jax.numpy as jnp
from jax import lax
from jax.experimental import pallas as pl
from jax.experimental.pallas import tpu as pltpu
from jax.experimental.pallas import tpu_sc as plsc

_NC = 2
_NS = 16
_L = 16


def _mm_body(x_ref, w_ref, b_ref, o_ref):
    o_ref[...] = (
        jnp.dot(x_ref[...], w_ref[...], preferred_element_type=jnp.float32)
        + b_ref[...]
    )


def _mm(x, W, b, bm=2000):
    n, d = x.shape
    f = W.shape[1]
    return pl.pallas_call(
        _mm_body,
        grid=(n // bm,),
        in_specs=[
            pl.BlockSpec((bm, d), lambda i: (i, 0)),
            pl.BlockSpec((d, f), lambda i: (0, 0)),
            pl.BlockSpec((1, f), lambda i: (0, 0)),
        ],
        out_specs=pl.BlockSpec((bm, f), lambda i: (i, 0)),
        out_shape=jax.ShapeDtypeStruct((n, f), jnp.float32),
    )(x, W, b.reshape(1, f))


def _mm_fused_body(p_ref, w_ref, b_ref, o_ref):
    h = jax.nn.relu(p_ref[0] + p_ref[1])
    o_ref[...] = (
        jnp.dot(h, w_ref[...], preferred_element_type=jnp.float32) + b_ref[...]
    )


def _mm_fused(parts, W, b, bm=2000):
    _, n, d = parts.shape
    f = W.shape[1]
    return pl.pallas_call(
        _mm_fused_body,
        grid=(n // bm,),
        in_specs=[
            pl.BlockSpec((2, bm, d), lambda i: (0, i, 0)),
            pl.BlockSpec((d, f), lambda i: (0, 0)),
            pl.BlockSpec((1, f), lambda i: (0, 0)),
        ],
        out_specs=pl.BlockSpec((bm, f), lambda i: (i, 0)),
        out_shape=jax.ShapeDtypeStruct((n, f), jnp.float32),
    )(parts, W, b.reshape(1, f))


def _make_add_body(f_out):
    def _add_body(p_ref, o_ref):
        o_ref[...] = p_ref[0, :, :f_out] + p_ref[1, :, :f_out]
    return _add_body


def _add_parts(parts, f_out, bm=2000):
    _, n, f = parts.shape
    return pl.pallas_call(
        _make_add_body(f_out),
        grid=(n // bm,),
        in_specs=[pl.BlockSpec((2, bm, f), lambda i: (0, i, 0))],
        out_specs=pl.BlockSpec((bm, f_out), lambda i: (i, 0)),
        out_shape=jax.ShapeDtypeStruct((n, f_out), jnp.float32),
    )(parts)


@functools.lru_cache(maxsize=None)
def _make_scatter(n, e, f, fs=None, b=80, nb=4):
    fs = f if fs is None else fs
    nw = _NC * _NS
    epw = e // nw
    nc_ = epw // b
    nz = n // b
    dpk = nb - 1
    dg = nb - 2
    assert epw % b == 0 and b % _L == 0 and nc_ >= nb + 1 and nb >= 3
    assert n % b == 0 and b % 8 == 0 and f % _L == 0 and fs % _L == 0

    mesh = plsc.VectorSubcoreMesh(core_axis_name="c", subcore_axis_name="s")

    @functools.partial(
        pl.kernel,
        out_type=jax.ShapeDtypeStruct((_NC, n, f), jnp.float32),
        mesh=mesh,
        scratch_types=(
            [pltpu.VMEM_SHARED((n, f), jnp.float32)]
            + [pltpu.VMEM((2, b), jnp.int32) for _ in range(nb)]
            + [pltpu.VMEM((b,), jnp.float32) for _ in range(nb)]
            + [pltpu.VMEM((b, f), jnp.float32) for _ in range(nb)]
            + [pltpu.SemaphoreType.DMA for _ in range(4 * nb)]
        ),
    )
    def scatter_kernel(h_hbm, pk_hbm, vals_hbm, out_hbm, acc, *bufs):
        pks = bufs[:nb]
        vals = bufs[nb:2 * nb]
        rows = bufs[2 * nb:3 * nb]
        psem = bufs[3 * nb:4 * nb]
        vsem = bufs[4 * nb:5 * nb]
        gsem = bufs[5 * nb:6 * nb]
        ssem = bufs[6 * nb:7 * nb]
        c = lax.axis_index("c")
        s = lax.axis_index("s")
        wid = c * _NS + s
        cbase = wid * nc_

        def start_pk(ci, s_):
            pltpu.async_copy(pk_hbm.at[cbase + ci], pks[s_], psem[s_])
            pltpu.async_copy(vals_hbm.at[pl.ds((cbase + ci) * b, b)],
                             vals[s_], vsem[s_])

        def wait_pk(s_):
            pltpu.make_async_copy(pk_hbm.at[0], pks[s_], psem[s_]).wait()
            pltpu.make_async_copy(vals_hbm.at[pl.ds(0, b)], vals[s_],
                                  vsem[s_]).wait()

        def start_gather(s_):
            pltpu.async_copy(h_hbm.at[pks[s_].at[0]], rows[s_], gsem[s_])

        def wait_gather(s_):
            pltpu.make_async_copy(h_hbm.at[pks[s_].at[0]], rows[s_],
                                  gsem[s_]).wait()

        def start_scatter(s_):
            pltpu.async_copy(rows[s_], acc.at[pks[s_].at[1]], ssem[s_],
                             add=True)

        def wait_scatter(s_):
            pltpu.make_async_copy(rows[s_], acc.at[pks[s_].at[1]],
                                  ssem[s_]).wait()

        def scale(s_):
            vals_ref = vals[s_]
            rows_ref = rows[s_]

            def body(g, carry):
                vvec = vals_ref[pl.ds(g * _L, _L)]
                for t in range(_L):
                    row = g * _L + t
                    v = vvec[t]
                    for jf in range(fs // _L):
                        sl = pl.ds(jf * _L, _L)
                        rows_ref[row, sl] = rows_ref[row, sl] * v
                return carry

            lax.fori_loop(0, b // _L, body, 0)

        zeros = jnp.zeros((_L,), jnp.float32)

        def zrow(i, carry):
            for j in range(f // _L):
                rows[0][i, pl.ds(j * _L, _L)] = zeros
            return carry

        lax.fori_loop(0, b, zrow, 0)
        for r in range((nz + _NS - 1) // _NS):
            kc = r * _NS + s

            @pl.when(kc < nz)
            def _():
                pltpu.sync_copy(rows[0], acc.at[pl.ds(kc * b, b)])

        plsc.subcore_barrier()

        for k in range(dpk):
            start_pk(k, k)
        for k in range(dg):
            wait_pk(k)
            start_gather(k)

        def rotation(i, carry):
            for j in range(nb):
                ci = i * nb + j

                @pl.when((ci + dpk < nc_) & (ci >= 1))
                def _():
                    wait_scatter((j + dpk) % nb)

                @pl.when(ci + dpk < nc_)
                def _():
                    start_pk(ci + dpk, (j + dpk) % nb)

                @pl.when(ci + dg < nc_)
                def _():
                    wait_pk((j + dg) % nb)
                    start_gather((j + dg) % nb)

                @pl.when(ci < nc_)
                def _():
                    wait_gather(j)
                    scale(j)
                    start_scatter(j)
            return carry

        lax.fori_loop(0, (nc_ + nb - 1) // nb, rotation, 0)
        for j in range(nb):
            wait_scatter(j)
        plsc.subcore_barrier()

        for r in range((nz + _NS - 1) // _NS):
            kc = r * _NS + s

            @pl.when(kc < nz)
            def _():
                pltpu.sync_copy(acc.at[pl.ds(kc * b, b)],
                                out_hbm.at[c, pl.ds(kc * b, b)])

    return scatter_kernel


def kernel(x, edge_index, edge_vals, W1, b1, W2, b2, W3, b3):
    n = x.shape[0]
    e = edge_vals.shape[0]
    h = W2.shape[0]
    c_out = W3.shape[1]
    src = edge_index[1]
    dst = edge_index[0]

    b = 80
    g = e // b
    packed = jnp.stack([src.reshape(g, b), dst.reshape(g, b)], axis=1)

    scatter_h = _make_scatter(n, e, h, b=b)
    scatter_l = _make_scatter(n, e, h, fs=c_out, b=b)

    W3p = jnp.pad(W3, ((0, 0), (0, h - c_out)))
    b3p = jnp.pad(b3, (0, h - c_out))

    a1 = _mm(x, W1, b1)
    p1 = scatter_h(a1, packed, edge_vals)
    a2 = _mm_fused(p1, W2, b2)
    p2 = scatter_h(a2, packed, edge_vals)
    a3 = _mm_fused(p2, W3p, b3p)
    p3 = scatter_l(a3, packed, edge_vals)
    return _add_parts(p3, c_out)
